# baseline (speedup 1.0000x reference)
import functools

import jax
import jax.numpy as jnp
from jax import lax
from jax.experimental import pallas as pl
from jax.experimental.pallas import tpu as pltpu
from jax.experimental.pallas import tpu_sc as plsc

N_NODES = 10000
N_EDGES = 320000
D = 128

NC = 2
NS = 16
NW = NC * NS

K = 128
CW = 80
C = CW * 128 // K
E_PAD = NW * CW * 128
NPAD = 10240
RPT = NPAD // NS


@functools.partial(
    pl.kernel,
    mesh=plsc.VectorSubcoreMesh(core_axis_name="c", subcore_axis_name="s"),
    out_type=jax.ShapeDtypeStruct((NC, NPAD, D), jnp.float32),
    scratch_types=[
        pltpu.VMEM((CW, 128), jnp.int32),
        pltpu.VMEM((CW, 128), jnp.int32),
        pltpu.VMEM((K, D), jnp.float32),
        pltpu.VMEM_SHARED((NPAD, D), jnp.float32),
        pltpu.SemaphoreType.DMA,
    ],
)
def _sc_aggregate(x_hbm, src_hbm, dst_hbm, zero_hbm, out_hbm,
                  src_v, dst_v, rows_v, agg_sh, sem):
    c = lax.axis_index("c")
    s = lax.axis_index("s")
    wid = s * NC + c

    pltpu.sync_copy(src_hbm.at[wid], src_v)
    pltpu.sync_copy(dst_hbm.at[wid], dst_v)
    pltpu.sync_copy(zero_hbm, agg_sh.at[pl.ds(s * RPT, RPT)])
    plsc.subcore_barrier()

    def body(j, carry):
        pltpu.async_copy(x_hbm.at[src_v.at[j]], rows_v, sem).wait()
        pltpu.sync_copy(rows_v, agg_sh.at[dst_v.at[j]], add=True)
        return carry

    lax.fori_loop(0, C, body, 0)
    plsc.subcore_barrier()

    pltpu.sync_copy(agg_sh.at[pl.ds(s * RPT, RPT)],
                    out_hbm.at[c, pl.ds(s * RPT, RPT)])


def _tc_linear_body(p_ref, wt_ref, b_ref, o_ref):
    agg = p_ref[0] + p_ref[1]
    out = jnp.dot(agg, wt_ref[...], preferred_element_type=jnp.float32)
    o_ref[...] = jnp.maximum(out + b_ref[...], 0.0)


_TC_ROWS = 2000


def _tc_linear(partials, wt, b2):
    return pl.pallas_call(
        _tc_linear_body,
        grid=(N_NODES // _TC_ROWS,),
        in_specs=[
            pl.BlockSpec((NC, _TC_ROWS, D), lambda i: (0, i, 0)),
            pl.BlockSpec((D, D), lambda i: (0, 0)),
            pl.BlockSpec((1, D), lambda i: (0, 0)),
        ],
        out_specs=pl.BlockSpec((_TC_ROWS, D), lambda i: (i, 0)),
        out_shape=jax.ShapeDtypeStruct((N_NODES, D), jnp.float32),
    )(partials, wt, b2)


def kernel(x, adj, W, b):
    adj32 = adj.astype(jnp.int32)
    epw = N_EDGES // NW
    ppw = E_PAD // NW - epw
    pad_src = jnp.zeros((NW, ppw), jnp.int32)
    pad_dst = jnp.broadcast_to(N_NODES + jnp.arange(ppw, dtype=jnp.int32),
                               (NW, ppw))
    src = jnp.concatenate([adj32[1].reshape(NW, epw), pad_src],
                          axis=1).reshape(NW, CW, 128)
    dst = jnp.concatenate([adj32[0].reshape(NW, epw), pad_dst],
                          axis=1).reshape(NW, CW, 128)
    zeros = jnp.zeros((RPT, D), jnp.float32)
    partials = _sc_aggregate(x, src, dst, zeros)
    return _tc_linear(partials, W.T, b.reshape(1, D))


# speedup vs baseline: 3.0792x; 3.0792x over previous
import functools

import jax
import jax.numpy as jnp
from jax import lax
from jax.experimental import pallas as pl
from jax.experimental.pallas import tpu as pltpu
from jax.experimental.pallas import tpu_sc as plsc

N_NODES = 10000
N_EDGES = 320000
D = 128

NC = 2
NS = 16
NW = NC * NS

K = 80
EPW = N_EDGES // NW
C = EPW // K
NPAD = 10240
RPT = NPAD // NS


@functools.partial(
    pl.kernel,
    mesh=plsc.VectorSubcoreMesh(core_axis_name="c", subcore_axis_name="s"),
    out_type=jax.ShapeDtypeStruct((NC, NPAD, D), jnp.float32),
    scratch_types=[
        pltpu.VMEM((EPW,), jnp.int32),
        pltpu.VMEM((C, 128), jnp.int32),
        pltpu.VMEM((2 * K, D), jnp.float32),
        pltpu.VMEM_SHARED((NPAD, D), jnp.float32),
        pltpu.SemaphoreType.DMA,
    ],
)
def _sc_aggregate(x_hbm, src_hbm, dst_hbm, zero_hbm, out_hbm,
                  src_v, dst_v, rows_v, agg_sh, sem):
    c = lax.axis_index("c")
    s = lax.axis_index("s")
    wid = s * NC + c

    pltpu.sync_copy(src_hbm.at[wid, 0], src_v)
    pltpu.sync_copy(dst_hbm.at[wid], dst_v)
    pltpu.sync_copy(zero_hbm, agg_sh.at[pl.ds(s * RPT, RPT)])
    plsc.subcore_barrier()

    def gather(j, slot):
        pltpu.async_copy(x_hbm.at[src_v.at[pl.ds(j * K, K)]],
                         rows_v.at[pl.ds(slot * K, K)], sem)

    def wait(slot):
        pltpu.make_async_copy(x_hbm.at[src_v.at[pl.ds(0, K)]],
                              rows_v.at[pl.ds(slot * K, K)], sem).wait()

    def scatter(j, slot):
        pltpu.sync_copy(rows_v.at[pl.ds(slot * K, K)],
                        agg_sh.at[dst_v.at[j, pl.ds(0, K)]], add=True)

    gather(0, 0)

    def body(jj, carry):
        j = 2 * jj
        gather(j + 1, 1)
        wait(0)
        scatter(j, 0)
        gather(j + 2, 0)
        wait(1)
        scatter(j + 1, 1)
        return carry

    lax.fori_loop(0, (C - 1) // 2, body, 0)
    wait(0)
    scatter(C - 1, 0)
    plsc.subcore_barrier()

    pltpu.sync_copy(agg_sh.at[pl.ds(s * RPT, RPT)],
                    out_hbm.at[c, pl.ds(s * RPT, RPT)])


def _tc_linear_body(p_ref, wt_ref, b_ref, o_ref):
    agg = p_ref[0] + p_ref[1]
    out = jnp.dot(agg, wt_ref[...], preferred_element_type=jnp.float32)
    o_ref[...] = jnp.maximum(out + b_ref[...], 0.0)


_TC_ROWS = 2000


def _tc_linear(partials, wt, b2):
    return pl.pallas_call(
        _tc_linear_body,
        grid=(N_NODES // _TC_ROWS,),
        in_specs=[
            pl.BlockSpec((NC, _TC_ROWS, D), lambda i: (0, i, 0)),
            pl.BlockSpec((D, D), lambda i: (0, 0)),
            pl.BlockSpec((1, D), lambda i: (0, 0)),
        ],
        out_specs=pl.BlockSpec((_TC_ROWS, D), lambda i: (i, 0)),
        out_shape=jax.ShapeDtypeStruct((N_NODES, D), jnp.float32),
    )(partials, wt, b2)


def kernel(x, adj, W, b):
    adj32 = adj.astype(jnp.int32)
    src = adj32[1].reshape(NW, 1, EPW)
    dst = jnp.pad(adj32[0].reshape(NW, C, K), ((0, 0), (0, 0), (0, 128 - K)))
    zeros = jnp.zeros((RPT, D), jnp.float32)
    partials = _sc_aggregate(x, src, dst, zeros)
    return _tc_linear(partials, W.T, b.reshape(1, D))


# speedup vs baseline: 3.0824x; 1.0010x over previous
import functools

import jax
import jax.numpy as jnp
from jax import lax
from jax.experimental import pallas as pl
from jax.experimental.pallas import tpu as pltpu
from jax.experimental.pallas import tpu_sc as plsc

N_NODES = 10000
N_EDGES = 320000
D = 128

NC = 2
NS = 16
NW = NC * NS

K = 80
EPW = N_EDGES // NW
C = EPW // K
NPAD = 10240
RPT = NPAD // NS


@functools.partial(
    pl.kernel,
    mesh=plsc.VectorSubcoreMesh(core_axis_name="c", subcore_axis_name="s"),
    out_type=jax.ShapeDtypeStruct((NC, NPAD, D), jnp.float32),
    scratch_types=[
        pltpu.VMEM((EPW,), jnp.int32),
        pltpu.VMEM((C, K), jnp.int32),
        pltpu.VMEM((2 * K, D), jnp.float32),
        pltpu.VMEM_SHARED((NPAD, D), jnp.float32),
        pltpu.SemaphoreType.DMA,
    ],
)
def _sc_aggregate(x_hbm, src_hbm, dst_hbm, zero_hbm, out_hbm,
                  src_v, dst_v, rows_v, agg_sh, sem):
    c = lax.axis_index("c")
    s = lax.axis_index("s")
    wid = s * NC + c

    pltpu.sync_copy(src_hbm.at[wid, 0], src_v)
    pltpu.sync_copy(dst_hbm.at[wid], dst_v)
    pltpu.sync_copy(zero_hbm, agg_sh.at[pl.ds(s * RPT, RPT)])
    plsc.subcore_barrier()

    def gather(j, slot):
        pltpu.async_copy(x_hbm.at[src_v.at[pl.ds(j * K, K)]],
                         rows_v.at[pl.ds(slot * K, K)], sem)

    def wait(slot):
        pltpu.make_async_copy(x_hbm.at[src_v.at[pl.ds(0, K)]],
                              rows_v.at[pl.ds(slot * K, K)], sem).wait()

    def scatter(j, slot):
        pltpu.sync_copy(rows_v.at[pl.ds(slot * K, K)],
                        agg_sh.at[dst_v.at[j]], add=True)

    gather(0, 0)

    def body(jj, carry):
        j = 2 * jj
        gather(j + 1, 1)
        wait(0)
        scatter(j, 0)
        gather(j + 2, 0)
        wait(1)
        scatter(j + 1, 1)
        return carry

    lax.fori_loop(0, (C - 1) // 2, body, 0)
    wait(0)
    scatter(C - 1, 0)
    plsc.subcore_barrier()

    pltpu.sync_copy(agg_sh.at[pl.ds(s * RPT, RPT)],
                    out_hbm.at[c, pl.ds(s * RPT, RPT)])


def _tc_linear_body(p_ref, wt_ref, b_ref, o_ref):
    agg = p_ref[0] + p_ref[1]
    out = jnp.dot(agg, wt_ref[...], preferred_element_type=jnp.float32)
    o_ref[...] = jnp.maximum(out + b_ref[...], 0.0)


_TC_ROWS = 2000


def _tc_linear(partials, wt, b2):
    return pl.pallas_call(
        _tc_linear_body,
        grid=(N_NODES // _TC_ROWS,),
        in_specs=[
            pl.BlockSpec((NC, _TC_ROWS, D), lambda i: (0, i, 0)),
            pl.BlockSpec((D, D), lambda i: (0, 0)),
            pl.BlockSpec((1, D), lambda i: (0, 0)),
        ],
        out_specs=pl.BlockSpec((_TC_ROWS, D), lambda i: (i, 0)),
        out_shape=jax.ShapeDtypeStruct((N_NODES, D), jnp.float32),
    )(partials, wt, b2)


def kernel(x, adj, W, b):
    adj32 = adj.astype(jnp.int32)
    src = adj32[1].reshape(NW, 1, EPW)
    dst = adj32[0].reshape(NW, C, K)
    zeros = jnp.zeros((RPT, D), jnp.float32)
    partials = _sc_aggregate(x, src, dst, zeros)
    return _tc_linear(partials, W.T, b.reshape(1, D))


# speedup vs baseline: 3.1074x; 1.0081x over previous
import functools

import jax
import jax.numpy as jnp
from jax import lax
from jax.experimental import pallas as pl
from jax.experimental.pallas import tpu as pltpu
from jax.experimental.pallas import tpu_sc as plsc

N_NODES = 10000
N_EDGES = 320000
D = 128

NC = 2
NS = 16
NW = NC * NS

K = 80
EPW = N_EDGES // NW
C = EPW // K
NPAD = 10240
RPT = NPAD // NS


@functools.partial(
    pl.kernel,
    mesh=plsc.VectorSubcoreMesh(core_axis_name="c", subcore_axis_name="s"),
    out_type=jax.ShapeDtypeStruct((NC, NPAD, D), jnp.float32),
    scratch_types=[
        pltpu.VMEM((EPW,), jnp.int32),
        pltpu.VMEM((C, K), jnp.int32),
        pltpu.VMEM((2 * K, D), jnp.float32),
        pltpu.VMEM_SHARED((NPAD, D), jnp.float32),
        pltpu.SemaphoreType.DMA,
        pltpu.SemaphoreType.DMA,
    ],
)
def _sc_aggregate(x_hbm, src_hbm, dst_hbm, zero_hbm, out_hbm,
                  src_v, dst_v, rows_v, agg_sh, sem, zsem):
    c = lax.axis_index("c")
    s = lax.axis_index("s")
    wid = s * NC + c

    zcp = pltpu.async_copy(zero_hbm.at[pl.ds(s * RPT, RPT)],
                           agg_sh.at[pl.ds(s * RPT, RPT)], zsem)
    pltpu.sync_copy(src_hbm.at[wid, 0], src_v)
    pltpu.sync_copy(dst_hbm.at[wid], dst_v)
    zcp.wait()
    plsc.subcore_barrier()

    def gather(j, slot):
        pltpu.async_copy(x_hbm.at[src_v.at[pl.ds(j * K, K)]],
                         rows_v.at[pl.ds(slot * K, K)], sem)

    def wait(slot):
        pltpu.make_async_copy(x_hbm.at[src_v.at[pl.ds(0, K)]],
                              rows_v.at[pl.ds(slot * K, K)], sem).wait()

    def scatter(j, slot):
        pltpu.sync_copy(rows_v.at[pl.ds(slot * K, K)],
                        agg_sh.at[dst_v.at[j]], add=True)

    gather(0, 0)

    def body(jj, carry):
        j = 2 * jj
        gather(j + 1, 1)
        wait(0)
        scatter(j, 0)
        gather(j + 2, 0)
        wait(1)
        scatter(j + 1, 1)
        return carry

    lax.fori_loop(0, (C - 1) // 2, body, 0)
    wait(0)
    scatter(C - 1, 0)
    plsc.subcore_barrier()

    pltpu.sync_copy(agg_sh.at[pl.ds(s * RPT, RPT)],
                    out_hbm.at[c, pl.ds(s * RPT, RPT)])


def _tc_linear_body(p_ref, wt_ref, b_ref, o_ref):
    agg = p_ref[0] + p_ref[1]
    out = jnp.dot(agg, wt_ref[...], preferred_element_type=jnp.float32)
    o_ref[...] = jnp.maximum(out + b_ref[...], 0.0)


_TC_ROWS = 2000


def _tc_linear(partials, wt, b2):
    return pl.pallas_call(
        _tc_linear_body,
        grid=(N_NODES // _TC_ROWS,),
        in_specs=[
            pl.BlockSpec((NC, _TC_ROWS, D), lambda i: (0, i, 0)),
            pl.BlockSpec((D, D), lambda i: (0, 0)),
            pl.BlockSpec((1, D), lambda i: (0, 0)),
        ],
        out_specs=pl.BlockSpec((_TC_ROWS, D), lambda i: (i, 0)),
        out_shape=jax.ShapeDtypeStruct((N_NODES, D), jnp.float32),
    )(partials, wt, b2)


def kernel(x, adj, W, b):
    adj32 = adj.astype(jnp.int32)
    src = adj32[1].reshape(NW, 1, EPW)
    dst = adj32[0].reshape(NW, C, K)
    zeros = jnp.zeros((NPAD, D), jnp.float32)
    partials = _sc_aggregate(x, src, dst, zeros)
    return _tc_linear(partials, W.T, b.reshape(1, D))
